# baseline (speedup 1.0000x reference)
import functools
import math

import jax
import jax.numpy as jnp
from jax import lax
from jax.experimental import pallas as pl
from jax.experimental.pallas import tpu as pltpu
from jax.experimental.pallas import tpu_sc as plsc

N = 100000
E = 1600000
NODE_DIM = 128
NUM_BASIS = 20
CUTOFF = 5.0

_NC = 2
_NS = 16
_NW = _NC * _NS
_L = 16

_BPW = 3128
_GCHUNK = 512
_NGCHUNK = 7
_N_PAD = _BPW * _NW

_EPW = E // _NW
_ECHUNK = 2000
_NECHUNK = _EPW // _ECHUNK
_EVREGS = _ECHUNK // _L

_PI = math.pi
_A = math.sqrt(2.0 / CUTOFF)

_S3 = -1.0 / 6.0
_S5 = 1.0 / 120.0
_S7 = -1.0 / 5040.0
_S9 = 1.0 / 362880.0
_C2 = -1.0 / 2.0
_C4 = 1.0 / 24.0
_C6 = -1.0 / 720.0
_C8 = 1.0 / 40320.0
_C10 = -1.0 / 3628800.0


def _sc_kernel(idx_hbm, table_hbm, d_hbm, ev_hbm,
               out_hbm, rbf_hbm, fcut_hbm, uvec_hbm,
               idx_v, rows_v, d_v, ev_v, rbf_v, fcut_v, inv_v, sem):
    wid = lax.axis_index("s") * _NC + lax.axis_index("c")

    gbase = wid * _BPW
    gcount = jnp.where(wid == _NW - 1, N - (_NW - 1) * _BPW, _BPW)
    pltpu.sync_copy(idx_hbm.at[pl.ds(gbase, _BPW)], idx_v)
    for i in range(_NGCHUNK):
        s = jnp.minimum(i * _GCHUNK, gcount - _GCHUNK)
        pltpu.async_copy(
            table_hbm.at[idx_v.at[pl.ds(s, _GCHUNK)]], rows_v, sem
        ).wait()
        pltpu.sync_copy(rows_v, out_hbm.at[pl.ds(gbase + s, _GCHUNK)])

    ebase = wid * _EPW
    lane = lax.iota(jnp.int32, _L)

    def chunk_body(c, _):
        e0 = ebase + c * _ECHUNK
        pltpu.sync_copy(d_hbm.at[pl.ds(e0, _ECHUNK)], d_v)
        pltpu.sync_copy(ev_hbm.at[pl.ds(3 * e0, 3 * _ECHUNK)], ev_v)

        def vreg_body(j, _):
            d = d_v[pl.ds(j * _L, _L)]
            theta = d * (_PI / CUTOFF)
            half = _PI / 2.0
            dev = theta - half
            phi = half - jnp.abs(dev)
            p2 = phi * phi
            sphi = phi * (1.0 + p2 * (_S3 + p2 * (_S5 + p2 * (_S7 + p2 * _S9))))
            cphi = 1.0 + p2 * (_C2 + p2 * (_C4 + p2 * (_C6 + p2 * (_C8 + p2 * _C10))))
            cth = jnp.where(dev > 0.0, -cphi, cphi)
            inv_d = 1.0 / d
            fcut = jnp.where(d < CUTOFF, 0.5 * (cth + 1.0), 0.0)
            fcut_v[pl.ds(j * _L, _L)] = fcut
            inv_v[pl.ds(j * _L, _L)] = inv_d
            base20 = (j * _L + lane) * NUM_BASIS
            twoc = cth + cth
            rkm1 = sphi * (_A * inv_d)
            plsc.store_scatter(rbf_v, [base20], rkm1)
            rk = twoc * rkm1
            plsc.store_scatter(rbf_v, [base20 + 1], rk)
            for k in range(2, NUM_BASIS):
                rkm1, rk = rk, twoc * rk - rkm1
                plsc.store_scatter(rbf_v, [base20 + k], rk)
            return _

        lax.fori_loop(0, _EVREGS, vreg_body, 0)

        def uvec_body(j, _):
            base3 = (j * _L + lane) * 3
            w = inv_v[pl.ds(j * _L, _L)]
            for comp in range(3):
                v = plsc.load_gather(ev_v, [base3 + comp])
                plsc.store_scatter(ev_v, [base3 + comp], v * w)
            return _

        lax.fori_loop(0, _EVREGS, uvec_body, 0)

        pltpu.sync_copy(rbf_v, rbf_hbm.at[pl.ds(NUM_BASIS * e0, NUM_BASIS * _ECHUNK)])
        pltpu.sync_copy(fcut_v, fcut_hbm.at[pl.ds(e0, _ECHUNK)])
        pltpu.sync_copy(ev_v, uvec_hbm.at[pl.ds(3 * e0, 3 * _ECHUNK)])
        return _

    lax.fori_loop(0, _NECHUNK, chunk_body, 0)


def kernel(atomic_numbers, edge_vector, edge_length, embed_table):
    idx = jnp.pad(atomic_numbers.astype(jnp.int32), (0, _N_PAD - N))
    ev_flat = edge_vector.reshape(3 * E)
    mesh = plsc.VectorSubcoreMesh(core_axis_name="c", subcore_axis_name="s")
    kern = functools.partial(
        pl.kernel,
        mesh=mesh,
        compiler_params=pltpu.CompilerParams(needs_layout_passes=False),
        out_type=(
            jax.ShapeDtypeStruct((N, NODE_DIM), jnp.float32),
            jax.ShapeDtypeStruct((NUM_BASIS * E,), jnp.float32),
            jax.ShapeDtypeStruct((E,), jnp.float32),
            jax.ShapeDtypeStruct((3 * E,), jnp.float32),
        ),
        scratch_types=[
            pltpu.VMEM((_BPW,), jnp.int32),
            pltpu.VMEM((_GCHUNK, NODE_DIM), jnp.float32),
            pltpu.VMEM((_ECHUNK,), jnp.float32),
            pltpu.VMEM((3 * _ECHUNK,), jnp.float32),
            pltpu.VMEM((NUM_BASIS * _ECHUNK,), jnp.float32),
            pltpu.VMEM((_ECHUNK,), jnp.float32),
            pltpu.VMEM((_ECHUNK,), jnp.float32),
            pltpu.SemaphoreType.DMA,
        ],
    )(_sc_kernel)
    node_invariant, rbf_flat, fcut_flat, uvec_flat = kern(
        idx, embed_table, edge_length, ev_flat
    )
    rbf = rbf_flat.reshape(E, NUM_BASIS)
    fcut = fcut_flat.reshape(E, 1)
    uvec = uvec_flat.reshape(E, 3)
    node_equivariant = jnp.zeros((N, 3, NODE_DIM), dtype=jnp.float32)
    return (node_invariant, rbf, fcut, uvec, node_equivariant)


# speedup vs baseline: 24.0946x; 24.0946x over previous
import functools
import math

import jax
import jax.numpy as jnp
from jax import lax
from jax.experimental import pallas as pl
from jax.experimental.pallas import tpu as pltpu
from jax.experimental.pallas import tpu_sc as plsc

N = 100000
E = 1600000
NODE_DIM = 128
NUM_BASIS = 20
CUTOFF = 5.0

_NC = 2
_NS = 16
_NW = _NC * _NS
_BPW = 3128
_GCHUNK = 512
_NGCHUNK = 7
_N_PAD = _BPW * _NW


def _sc_gather_kernel(idx_hbm, table_hbm, out_hbm, idx_v, rows_v, sem):
    wid = lax.axis_index("s") * _NC + lax.axis_index("c")
    base = wid * _BPW
    count = jnp.where(wid == _NW - 1, N - (_NW - 1) * _BPW, _BPW)
    pltpu.sync_copy(idx_hbm.at[pl.ds(base, _BPW)], idx_v)
    for i in range(_NGCHUNK):
        s = jnp.minimum(i * _GCHUNK, count - _GCHUNK)
        pltpu.async_copy(
            table_hbm.at[idx_v.at[pl.ds(s, _GCHUNK)]], rows_v, sem
        ).wait()
        pltpu.sync_copy(rows_v, out_hbm.at[pl.ds(base + s, _GCHUNK)])


def _sc_gather(atomic_numbers, embed_table):
    idx = jnp.pad(atomic_numbers.astype(jnp.int32), (0, _N_PAD - N))
    mesh = plsc.VectorSubcoreMesh(core_axis_name="c", subcore_axis_name="s")
    kern = functools.partial(
        pl.kernel,
        mesh=mesh,
        compiler_params=pltpu.CompilerParams(needs_layout_passes=False),
        out_type=jax.ShapeDtypeStruct((N, NODE_DIM), jnp.float32),
        scratch_types=[
            pltpu.VMEM((_BPW,), jnp.int32),
            pltpu.VMEM((_GCHUNK, NODE_DIM), jnp.float32),
            pltpu.SemaphoreType.DMA,
        ],
    )(_sc_gather_kernel)
    return kern(idx, embed_table)


_BE = 12800
_GRID = E // _BE

_PI = math.pi
_A = math.sqrt(2.0 / CUTOFF)

_S3 = -1.0 / 6.0
_S5 = 1.0 / 120.0
_S7 = -1.0 / 5040.0
_S9 = 1.0 / 362880.0


def _sin_reduced(ang):
    q = (ang * (1.0 / _PI)).astype(jnp.int32)
    r = ang - q.astype(jnp.float32) * _PI
    half = _PI / 2.0
    phi = half - jnp.abs(r - half)
    p2 = phi * phi
    s = phi * (1.0 + p2 * (_S3 + p2 * (_S5 + p2 * (_S7 + p2 * _S9))))
    return jnp.where((q & 1) == 0, s, -s)


def _tc_edge_kernel(d_ref, ev_ref, rbf_ref, fcut_ref, uvec_ref):
    d = d_ref[...].reshape(1, _BE)
    theta = d * (_PI / CUTOFF)
    inv_d = 1.0 / d
    w = _A * inv_d

    karr = (lax.broadcasted_iota(jnp.int32, (8, 1), 0) + 1).astype(jnp.float32)
    ang = karr * theta
    s8 = _sin_reduced(ang)
    c8 = _sin_reduced(ang + (_PI / 2.0))

    s8r = s8[7:8, :]
    c8r = c8[7:8, :]
    s16 = 2.0 * s8r * c8r
    c16 = 1.0 - 2.0 * s8r * s8r

    rbf_ref[0:8, :] = w * s8
    rbf_ref[8:16, :] = w * (s8r * c8 + c8r * s8)
    slab3 = w * (s16 * c8 + c16 * s8)
    rbf_ref[16:NUM_BASIS, :] = slab3[0 : NUM_BASIS - 16, :]

    c1 = c8[0:1, :]
    fcut_ref[...] = jnp.where(
        d < CUTOFF, 0.5 * (c1 + 1.0), 0.0
    ).reshape(1, 1, _BE)
    uvec_ref[...] = ev_ref[...] * inv_d


def _tc_edges(edge_vector, edge_length):
    d2 = edge_length.reshape(_GRID, 1, _BE)
    ev_t = edge_vector.T
    rbf_t, fcut2, uvec_t = pl.pallas_call(
        _tc_edge_kernel,
        grid=(_GRID,),
        in_specs=[
            pl.BlockSpec((1, 1, _BE), lambda i: (i, 0, 0)),
            pl.BlockSpec((3, _BE), lambda i: (0, i)),
        ],
        out_specs=[
            pl.BlockSpec((NUM_BASIS, _BE), lambda i: (0, i)),
            pl.BlockSpec((1, 1, _BE), lambda i: (i, 0, 0)),
            pl.BlockSpec((3, _BE), lambda i: (0, i)),
        ],
        out_shape=[
            jax.ShapeDtypeStruct((NUM_BASIS, E), jnp.float32),
            jax.ShapeDtypeStruct((_GRID, 1, _BE), jnp.float32),
            jax.ShapeDtypeStruct((3, E), jnp.float32),
        ],
    )(d2, ev_t)
    rbf = rbf_t.T
    fcut = fcut2.reshape(E, 1)
    uvec = uvec_t.T
    return rbf, fcut, uvec


def kernel(atomic_numbers, edge_vector, edge_length, embed_table):
    node_invariant = _sc_gather(atomic_numbers, embed_table)
    rbf, fcut, uvec = _tc_edges(edge_vector, edge_length)
    node_equivariant = jnp.zeros((N, 3, NODE_DIM), dtype=jnp.float32)
    return (node_invariant, rbf, fcut, uvec, node_equivariant)
